# speedup vs baseline: 3.8685x; 3.8685x over previous
import jax
import jax.numpy as jnp
from jax import lax
from jax.experimental import pallas as pl
from jax.experimental.pallas import tpu as pltpu

_BN_EPS = 1e-5
_H = 56
_W = 56
_S = _H * _W


def _bottleneck_kernel(x_ref, w1_ref, b1_ref, w2_ref, b2_ref, w3_ref, b3_ref,
                       o_ref):
    xb = x_ref[0]
    x16 = xb.astype(jnp.bfloat16)

    h1 = jnp.dot(w1_ref[...], x16, preferred_element_type=jnp.float32)
    h1 = jnp.maximum(h1 + b1_ref[...], 0.0).astype(jnp.bfloat16)

    mid = h1.shape[0]
    zpad = jnp.zeros((mid, _W + 1), jnp.bfloat16)
    hp = jnp.concatenate([zpad, h1, zpad], axis=1)
    col = lax.broadcasted_iota(jnp.int32, (1, _S), 1) % _W
    not_left = col != 0
    not_right = col != (_W - 1)
    taps = []
    for dy in (-1, 0, 1):
        for dx in (-1, 0, 1):
            off = _W + 1 + dy * _W + dx
            t = lax.slice(hp, (0, off), (mid, off + _S))
            if dx == -1:
                t = jnp.where(not_left, t, jnp.bfloat16(0))
            elif dx == 1:
                t = jnp.where(not_right, t, jnp.bfloat16(0))
            taps.append(t)
    slab = jnp.concatenate(taps, axis=0)
    h2 = jnp.dot(w2_ref[...], slab, preferred_element_type=jnp.float32)
    h2 = jnp.maximum(h2 + b2_ref[...], 0.0).astype(jnp.bfloat16)

    y = jnp.dot(w3_ref[...], h2, preferred_element_type=jnp.float32)
    y = y + b3_ref[...] + xb
    o_ref[0] = jnp.maximum(y, 0.0)


def _fold_bn(gamma, beta, mean, var):
    scale = gamma / jnp.sqrt(var + _BN_EPS)
    shift = beta - mean * scale
    return scale, shift


def kernel(x, w1, bn1_gamma, bn1_beta, bn1_mean, bn1_var,
           w2, bn2_gamma, bn2_beta, bn2_mean, bn2_var,
           w3, bn3_gamma, bn3_beta, bn3_mean, bn3_var):
    N, Cin, H, W = x.shape
    mid = w1.shape[-1]
    Cout = w3.shape[-1]

    s1, b1 = _fold_bn(bn1_gamma, bn1_beta, bn1_mean, bn1_var)
    s2, b2 = _fold_bn(bn2_gamma, bn2_beta, bn2_mean, bn2_var)
    s3, b3 = _fold_bn(bn3_gamma, bn3_beta, bn3_mean, bn3_var)

    w1t = (w1.reshape(Cin, mid) * s1[None, :]).T.astype(jnp.bfloat16)
    w2t = (w2 * s2[None, None, None, :]).reshape(9 * mid, mid).T.astype(
        jnp.bfloat16)
    w3t = (w3.reshape(mid, Cout) * s3[None, :]).T.astype(jnp.bfloat16)

    xr = x.reshape(N, Cin, H * W)

    out = pl.pallas_call(
        _bottleneck_kernel,
        grid=(N,),
        in_specs=[
            pl.BlockSpec((1, Cin, H * W), lambda i: (i, 0, 0)),
            pl.BlockSpec((mid, Cin), lambda i: (0, 0)),
            pl.BlockSpec((mid, 1), lambda i: (0, 0)),
            pl.BlockSpec((mid, 9 * mid), lambda i: (0, 0)),
            pl.BlockSpec((mid, 1), lambda i: (0, 0)),
            pl.BlockSpec((Cout, mid), lambda i: (0, 0)),
            pl.BlockSpec((Cout, 1), lambda i: (0, 0)),
        ],
        out_specs=pl.BlockSpec((1, Cout, H * W), lambda i: (i, 0, 0)),
        out_shape=jax.ShapeDtypeStruct((N, Cout, H * W), jnp.float32),
        compiler_params=pltpu.CompilerParams(
            dimension_semantics=("parallel",),
            vmem_limit_bytes=48 * 1024 * 1024),
    )(xr, w1t, b1.reshape(mid, 1), w2t, b2.reshape(mid, 1),
      w3t, b3.reshape(Cout, 1))

    return out.reshape(N, Cout, H, W)
